# baseline (speedup 1.0000x reference)
import jax
import jax.numpy as jnp
from jax.experimental import pallas as pl
from jax.experimental.pallas import tpu as pltpu

_CHUNK = 256
_N_STRIDED = 2


def _split(v):
    hi = v.astype(jnp.bfloat16)
    lo = (v - hi.astype(jnp.float32)).astype(jnp.bfloat16)
    return hi, lo


def _dot(a, b):
    return jax.lax.dot_general(
        a, b, (((1,), (0,)), ((), ())), preferred_element_type=jnp.float32
    )


def _bf(v):
    return v.astype(jnp.bfloat16)


def _net_kernel(x_ref, adj_hbm, W1a_ref, b1a_ref, W1b_ref, b1b_ref,
                W2a_ref, b2a_ref, W2b_ref, b2b_ref,
                W3a_ref, b3a_ref, W3b_ref, b3b_ref, out_ref,
                A_f32, A_bf, g1_buf, sem, sem2):
    N = adj_hbm.shape[0]
    F = x_ref.shape[1]
    n_chunks = N // _CHUNK
    n_contig = n_chunks - _N_STRIDED
    tail0 = n_contig * _CHUNK

    def chunk_copy(k):
        sl = pl.ds(k * _CHUNK, _CHUNK)
        return pltpu.make_async_copy(adj_hbm.at[sl, :], A_f32.at[sl, :], sem.at[k])

    def panel_copy(j, c):
        rs = pl.ds((n_contig + j) * _CHUNK, _CHUNK)
        cs = pl.ds(c * _CHUNK, _CHUNK)
        return pltpu.make_async_copy(
            adj_hbm.at[rs, cs], A_f32.at[rs, cs], sem2.at[c * _N_STRIDED + j])

    for k in range(n_contig):
        chunk_copy(k).start()
    for c in range(n_chunks):
        for j in range(_N_STRIDED):
            panel_copy(j, c).start()

    g = x_ref[...].T
    hi, lo = _split(g)
    s = jnp.concatenate([hi, lo], axis=0)
    s_tail = s[:, tail0:]
    w1a = _bf(W1a_ref[...].T)
    w1b = _bf(W1b_ref[...].T)
    w2a = _bf(W2a_ref[...].T)
    w2b = _bf(W2b_ref[...].T)
    w3a = _bf(W3a_ref[...].T)
    w3b = _bf(W3b_ref[...].T)
    b1a = b1a_ref[...].reshape(-1, 1)
    b1b = b1b_ref[...].reshape(-1, 1)
    b2a = b2a_ref[...].reshape(-1, 1)
    b2b = b2b_ref[...].reshape(-1, 1)
    b3a = b3a_ref[...].reshape(-1, 1)
    b3b = b3b_ref[...].reshape(-1, 1)

    acc2 = jnp.zeros((2 * F, N), jnp.float32)
    for k in range(n_contig):
        chunk_copy(k).wait()
        sl = slice(k * _CHUNK, (k + 1) * _CHUNK)
        a_k = _bf(A_f32[sl, :])
        A_bf[sl, :] = a_k
        acc2 = acc2 + _dot(s[:, sl], a_k)
    zbase = g + acc2[:F] + acc2[F:]

    agg2 = jnp.zeros((F, N), jnp.float32)
    for c in range(n_chunks):
        for j in range(_N_STRIDED):
            panel_copy(j, c).wait()
        cs = slice(c * _CHUNK, (c + 1) * _CHUNK)
        a_c = _bf(A_f32[tail0:, cs])
        A_bf[tail0:, cs] = a_c
        r = _dot(s_tail, a_c)
        z_c = zbase[:, cs] + r[:F] + r[F:]
        u_c = jnp.maximum(_dot(w1a, _bf(z_c)) + b1a, 0.0)
        g1_c = jnp.maximum(_dot(w1b, _bf(u_c)) + b1b, 0.0)
        g1_buf[:, cs] = g1_c
        if c < n_contig:
            agg2 = agg2 + _dot(_bf(g1_c), A_bf[cs, :])

    for c in range(n_contig, n_chunks):
        cs = slice(c * _CHUNK, (c + 1) * _CHUNK)
        agg2 = agg2 + _dot(_bf(g1_buf[:, cs]), A_bf[cs, :])
    z2 = g1_buf[...] + agg2
    u2 = jnp.maximum(_dot(w2a, _bf(z2)) + b2a, 0.0)
    g2 = jnp.maximum(jnp.maximum(_dot(w2b, _bf(u2)) + b2b, 0.0), 0.0)
    z3 = g2 + _dot(_bf(g2), A_bf[...])
    u3 = jnp.maximum(_dot(w3a, _bf(z3)) + b3a, 0.0)
    u3_mean = jnp.mean(u3, axis=1, keepdims=True)
    m_hi, m_lo = _split(u3_mean)
    out = _dot(w3b, m_hi) + _dot(w3b, m_lo) + b3b
    out_ref[...] = out.T


@jax.jit
def kernel(x, adj, W1a, b1a, W1b, b1b, W2a, b2a, W2b, b2b, W3a, b3a, W3b, b3b):
    N = adj.shape[0]
    H = W1b.shape[1]
    O = W3b.shape[1]
    vmem = pl.BlockSpec(memory_space=pltpu.MemorySpace.VMEM)
    return pl.pallas_call(
        _net_kernel,
        out_shape=jax.ShapeDtypeStruct((1, O), jnp.float32),
        in_specs=[vmem, pl.BlockSpec(memory_space=pltpu.MemorySpace.HBM)]
        + [vmem] * 12,
        scratch_shapes=[
            pltpu.VMEM((N, N), jnp.float32),
            pltpu.VMEM((N, N), jnp.bfloat16),
            pltpu.VMEM((H, N), jnp.float32),
            pltpu.SemaphoreType.DMA((N // _CHUNK - _N_STRIDED,)),
            pltpu.SemaphoreType.DMA(((N // _CHUNK) * _N_STRIDED,)),
        ],
        compiler_params=pltpu.CompilerParams(
            vmem_limit_bytes=100 * 1024 * 1024,
        ),
    )(x, adj, W1a, b1a, W1b, b1b, W2a, b2a, W2b, b2b, W3a, b3a, W3b, b3b)


# speedup vs baseline: 1.3298x; 1.3298x over previous
import jax
import jax.numpy as jnp
from jax.experimental import pallas as pl
from jax.experimental.pallas import tpu as pltpu

_CHUNK = 256


def _split(v):
    hi = v.astype(jnp.bfloat16)
    lo = (v - hi.astype(jnp.float32)).astype(jnp.bfloat16)
    return hi, lo


def _dot(a, b):
    return jax.lax.dot_general(
        a, b, (((1,), (0,)), ((), ())), preferred_element_type=jnp.float32
    )


def _net_kernel(x_ref, adj_hbm, W1a_ref, b1a_ref, W1b_ref, b1b_ref,
                W2a_ref, b2a_ref, W2b_ref, b2b_ref,
                W3a_ref, b3a_ref, W3b_ref, b3b_ref, out_ref,
                A_f32, A_bf, sem):
    N = adj_hbm.shape[0]
    F = x_ref.shape[1]
    n_chunks = N // _CHUNK

    def chunk_copy(k):
        sl = pl.ds(k * _CHUNK, _CHUNK)
        return pltpu.make_async_copy(adj_hbm.at[sl, :], A_f32.at[sl, :], sem.at[k])

    for k in range(n_chunks):
        chunk_copy(k).start()

    g = x_ref[...].T
    hi, lo = _split(g)
    s = jnp.concatenate([hi, lo], axis=0)
    w1a = W1a_ref[...].T.astype(jnp.bfloat16)
    w1b = W1b_ref[...].T.astype(jnp.bfloat16)
    w2a = W2a_ref[...].T.astype(jnp.bfloat16)
    w2b = W2b_ref[...].T.astype(jnp.bfloat16)
    w3a = W3a_ref[...].T.astype(jnp.bfloat16)
    w3b = W3b_ref[...].T.astype(jnp.bfloat16)
    b1a = b1a_ref[...].reshape(-1, 1)
    b1b = b1b_ref[...].reshape(-1, 1)
    b2a = b2a_ref[...].reshape(-1, 1)
    b2b = b2b_ref[...].reshape(-1, 1)
    b3a = b3a_ref[...].reshape(-1, 1)
    b3b = b3b_ref[...].reshape(-1, 1)

    acc2 = jnp.zeros((2 * F, N), jnp.float32)
    for k in range(n_chunks):
        chunk_copy(k).wait()
        sl = slice(k * _CHUNK, (k + 1) * _CHUNK)
        a_k = A_f32[sl, :].astype(jnp.bfloat16)
        A_bf[sl, :] = a_k
        acc2 = acc2 + _dot(s[:, sl], a_k)
    acc = acc2[:F] + acc2[F:]

    A = A_bf[...]

    def mlp_fast(z, wa, ba, wb, bb):
        u = jnp.maximum(_dot(wa, z.astype(jnp.bfloat16)) + ba, 0.0)
        return _dot(wb, u.astype(jnp.bfloat16)) + bb

    def agg_fast(t):
        return t + _dot(t.astype(jnp.bfloat16), A)

    g1 = jnp.maximum(mlp_fast(g + acc, w1a, b1a, w1b, b1b), 0.0)
    g2 = jnp.maximum(mlp_fast(agg_fast(g1), w2a, b2a, w2b, b2b), 0.0)
    u3 = jnp.maximum(
        _dot(w3a, agg_fast(g2).astype(jnp.bfloat16)) + b3a, 0.0)
    u3_mean = jnp.mean(u3, axis=1, keepdims=True)
    m_hi, m_lo = _split(u3_mean)
    out = _dot(w3b, m_hi) + _dot(w3b, m_lo) + b3b
    out_ref[...] = out.T


@jax.jit
def kernel(x, adj, W1a, b1a, W1b, b1b, W2a, b2a, W2b, b2b, W3a, b3a, W3b, b3b):
    N = adj.shape[0]
    O = W3b.shape[1]
    vmem = pl.BlockSpec(memory_space=pltpu.MemorySpace.VMEM)
    return pl.pallas_call(
        _net_kernel,
        out_shape=jax.ShapeDtypeStruct((1, O), jnp.float32),
        in_specs=[vmem, pl.BlockSpec(memory_space=pltpu.MemorySpace.HBM)]
        + [vmem] * 12,
        scratch_shapes=[
            pltpu.VMEM((N, N), jnp.float32),
            pltpu.VMEM((N, N), jnp.bfloat16),
            pltpu.SemaphoreType.DMA((N // _CHUNK,)),
        ],
        compiler_params=pltpu.CompilerParams(
            vmem_limit_bytes=100 * 1024 * 1024,
        ),
    )(x, adj, W1a, b1a, W1b, b1b, W2a, b2a, W2b, b2b, W3a, b3a, W3b, b3b)
